# baseline (speedup 1.0000x reference)
import functools

import jax
import jax.numpy as jnp
from jax import lax
from jax.experimental import pallas as pl
from jax.experimental.pallas import tpu as pltpu
from jax.experimental.pallas import tpu_sc as plsc

_TIME_INTERVAL = 86400.0
_N_TIME_INTERVAL = 100
_LANES = 16
_SUPER = 512


@functools.partial(jax.jit, static_argnums=(2, 3, 4))
def _sc_lookup(table_flat, ts_flat, n_rows, row_len, emb):
    n = n_rows * row_len
    info = plsc.get_sparse_core_info()
    nc, ns = info.num_cores, info.num_subcores
    nw = nc * ns
    per_w = n // nw
    n_super = per_w // _SUPER
    groups = _SUPER // _LANES

    mesh = plsc.VectorSubcoreMesh(core_axis_name="c", subcore_axis_name="s")

    @functools.partial(
        pl.kernel,
        mesh=mesh,
        out_type=jax.ShapeDtypeStruct((n * emb,), jnp.float32),
        scratch_types=[
            pltpu.VMEM((8 + per_w,), jnp.int32),
            pltpu.VMEM((table_flat.shape[0],), jnp.float32),
            pltpu.VMEM((2, _SUPER * emb), jnp.float32),
            pltpu.SemaphoreType.DMA((2,)),
        ],
        compiler_params=pltpu.CompilerParams(
            use_tc_tiling_on_sc=False, needs_layout_passes=False),
    )
    def k(table_hbm, ts_hbm, out_hbm, ts_v, table_v, rows_v, ssem):
        wid = lax.axis_index("s") * nc + lax.axis_index("c")
        base = wid * per_w
        pltpu.sync_copy(table_hbm, table_v)
        pltpu.sync_copy(ts_hbm.at[pl.ds(base, per_w)], ts_v.at[pl.ds(8, per_w)])
        iot = lax.iota(jnp.int32, _LANES)

        def run_groups(j, s):
          @plsc.parallel_loop(0, groups, unroll=2)
          def group_body(g):
            off = j * _SUPER + g * _LANES
            cur = ts_v[pl.ds(off + 8, _LANES)]
            prev = ts_v[pl.ds(off + 7, _LANES)]
            diff = (cur - prev).astype(jnp.float32)
            t = diff / _TIME_INTERVAL * float(_N_TIME_INTERVAL)
            iv = t.astype(jnp.int32)
            iv = jnp.minimum(jnp.maximum(iv, 0), _N_TIME_INTERVAL)
            rel = lax.rem(off + iot, jnp.int32(row_len))
            iv = jnp.where(rel == 0, 0, iv)
            src = iv * emb
            dst = (g * _LANES + iot) * emb
            batch = 8
            prev = None
            for cb in range(0, emb, batch):
                vs = [plsc.load_gather(table_v, [src + (cb + t)])
                      for t in range(batch)]
                if prev is not None:
                    pb, pvs = prev
                    for t in range(batch):
                        plsc.store_scatter(rows_v.at[s], [dst + (pb + t)], pvs[t])
                prev = (cb, vs)
            pb, pvs = prev
            for t in range(batch):
                plsc.store_scatter(rows_v.at[s], [dst + (pb + t)], pvs[t])

        def fire_store(j, s):
            pltpu.async_copy(
                rows_v.at[s],
                out_hbm.at[pl.ds((base + j * _SUPER) * emb, _SUPER * emb)],
                ssem.at[s],
            )

        def wait_store(s):
            pltpu.make_async_copy(
                rows_v.at[s],
                out_hbm.at[pl.ds(base * emb, _SUPER * emb)],
                ssem.at[s],
            ).wait()

        def body(i, carry):
            for s in (0, 1):
                j = i * 2 + s


                run_groups(j, s)
            return carry

        lax.fori_loop(0, n_super // 2, body, 0)

    return k(table_flat, ts_flat)


def kernel(inputs, timestamp, W, b):
    batch, max_len = timestamp.shape
    emb = W.shape[0]
    table = (W.T + b[None, :]).reshape(-1)
    out = _sc_lookup(table, timestamp.reshape(batch * max_len), batch, max_len, emb)
    return out.reshape(batch, max_len, emb)


# speedup vs baseline: 3.6693x; 3.6693x over previous
import functools

import jax
import jax.numpy as jnp
from jax import lax
from jax.experimental import pallas as pl
from jax.experimental.pallas import tpu as pltpu
from jax.experimental.pallas import tpu_sc as plsc

_TIME_INTERVAL = 86400.0
_N_TIME_INTERVAL = 100
_LANES = 16
_SUPER = 512


@functools.partial(jax.jit, static_argnums=(2, 3, 4))
def _sc_lookup(table_flat, ts_flat, n_rows, row_len, emb):
    n = n_rows * row_len
    info = plsc.get_sparse_core_info()
    nc, ns = info.num_cores, info.num_subcores
    nw = nc * ns
    per_w = n // nw
    n_super = per_w // _SUPER
    groups = _SUPER // _LANES
    nchunk = emb // _LANES

    mesh = plsc.VectorSubcoreMesh(core_axis_name="c", subcore_axis_name="s")

    @functools.partial(
        pl.kernel,
        mesh=mesh,
        out_type=jax.ShapeDtypeStruct((n * emb,), jnp.float32),
        scratch_types=[
            pltpu.VMEM((8 + per_w,), jnp.int32),
            pltpu.VMEM((table_flat.shape[0],), jnp.float32),
            pltpu.VMEM((2, _SUPER * emb), jnp.float32),
            pltpu.SemaphoreType.DMA((2,)),
        ],
        compiler_params=pltpu.CompilerParams(
            use_tc_tiling_on_sc=False, needs_layout_passes=False),
    )
    def k(table_hbm, ts_hbm, out_hbm, ts_v, table_v, rows_v, ssem):
        wid = lax.axis_index("s") * nc + lax.axis_index("c")
        base = wid * per_w
        pltpu.sync_copy(table_hbm, table_v)
        pltpu.sync_copy(ts_hbm.at[pl.ds(base, per_w)], ts_v.at[pl.ds(8, per_w)])
        iot = lax.iota(jnp.int32, _LANES)

        def run_groups(j, s):
          @plsc.parallel_loop(0, groups, unroll=2)
          def group_body(g):
            off = j * _SUPER + g * _LANES
            cur = ts_v[pl.ds(off + 8, _LANES)]
            prev = ts_v[pl.ds(off + 7, _LANES)]
            diff = (cur - prev).astype(jnp.float32)
            t = diff / _TIME_INTERVAL * float(_N_TIME_INTERVAL)
            iv = t.astype(jnp.int32)
            iv = jnp.minimum(jnp.maximum(iv, 0), _N_TIME_INTERVAL)
            rel = lax.rem(off + iot, jnp.int32(row_len))
            iv = jnp.where(rel == 0, 0, iv)
            src = iv * emb
            dstg = g * (_LANES * emb)
            rows_s = rows_v.at[s]
            for r0 in range(0, _LANES, 2):
                vals = []
                for r in (r0, r0 + 1):
                    tb = src[r]
                    vals.append([table_v[pl.ds(tb + c * _LANES, _LANES)]
                                 for c in range(nchunk)])
                for t_i, r in enumerate((r0, r0 + 1)):
                    for c in range(nchunk):
                        rows_s[pl.ds(dstg + r * emb + c * _LANES, _LANES)] = vals[t_i][c]

        def fire_store(j, s):
            pltpu.async_copy(
                rows_v.at[s],
                out_hbm.at[pl.ds((base + j * _SUPER) * emb, _SUPER * emb)],
                ssem.at[s],
            )

        def wait_store(s):
            pltpu.make_async_copy(
                rows_v.at[s],
                out_hbm.at[pl.ds(base * emb, _SUPER * emb)],
                ssem.at[s],
            ).wait()

        def body(i, carry):
            for s in (0, 1):
                j = i * 2 + s

                @pl.when(j >= 2)
                def _():
                    wait_store(s)

                run_groups(j, s)
                fire_store(j, s)
            return carry

        lax.fori_loop(0, n_super // 2, body, 0)
        wait_store(0)
        wait_store(1)

    return k(table_flat, ts_flat)


def kernel(inputs, timestamp, W, b):
    batch, max_len = timestamp.shape
    emb = W.shape[0]
    table = (W.T + b[None, :]).reshape(-1)
    out = _sc_lookup(table, timestamp.reshape(batch * max_len), batch, max_len, emb)
    return out.reshape(batch, max_len, emb)


# speedup vs baseline: 3.6719x; 1.0007x over previous
import functools

import jax
import jax.numpy as jnp
from jax import lax
from jax.experimental import pallas as pl
from jax.experimental.pallas import tpu as pltpu
from jax.experimental.pallas import tpu_sc as plsc

_TIME_INTERVAL = 86400.0
_N_TIME_INTERVAL = 100
_LANES = 16
_SUPER = 512


@functools.partial(jax.jit, static_argnums=(2, 3, 4))
def _sc_lookup(table_flat, ts_flat, n_rows, row_len, emb):
    n = n_rows * row_len
    info = plsc.get_sparse_core_info()
    nc, ns = info.num_cores, info.num_subcores
    nw = nc * ns
    per_w = n // nw
    n_super = per_w // _SUPER
    groups = _SUPER // _LANES
    nchunk = emb // _LANES

    mesh = plsc.VectorSubcoreMesh(core_axis_name="c", subcore_axis_name="s")

    @functools.partial(
        pl.kernel,
        mesh=mesh,
        out_type=jax.ShapeDtypeStruct((n, emb), jnp.float32),
        scratch_types=[
            pltpu.VMEM((8 + per_w,), jnp.int32),
            pltpu.VMEM((table_flat.shape[0],), jnp.float32),
            pltpu.VMEM((2, _SUPER, emb), jnp.float32),
            pltpu.SemaphoreType.DMA((2,)),
        ],
        compiler_params=pltpu.CompilerParams(
            use_tc_tiling_on_sc=False, needs_layout_passes=False),
    )
    def k(table_hbm, ts_hbm, out_hbm, ts_v, table_v, rows_v, ssem):
        wid = lax.axis_index("s") * nc + lax.axis_index("c")
        base = wid * per_w
        pltpu.sync_copy(table_hbm, table_v)
        pltpu.sync_copy(ts_hbm.at[pl.ds(base, per_w)], ts_v.at[pl.ds(8, per_w)])
        iot = lax.iota(jnp.int32, _LANES)

        def run_groups(j, s):
          @plsc.parallel_loop(0, groups, unroll=2)
          def group_body(g):
            off = j * _SUPER + g * _LANES
            cur = ts_v[pl.ds(off + 8, _LANES)]
            prev = ts_v[pl.ds(off + 7, _LANES)]
            diff = (cur - prev).astype(jnp.float32)
            t = diff / _TIME_INTERVAL * float(_N_TIME_INTERVAL)
            iv = t.astype(jnp.int32)
            iv = jnp.minimum(jnp.maximum(iv, 0), _N_TIME_INTERVAL)
            rel = lax.rem(off + iot, jnp.int32(row_len))
            iv = jnp.where(rel == 0, 0, iv)
            src = iv * emb
            rowg = g * _LANES
            for r0 in range(0, _LANES, 2):
                vals = []
                for r in (r0, r0 + 1):
                    tb = src[r]
                    vals.append([table_v[pl.ds(tb + c * _LANES, _LANES)]
                                 for c in range(nchunk)])
                for t_i, r in enumerate((r0, r0 + 1)):
                    row_ref = rows_v.at[s, rowg + r]
                    for c in range(nchunk):
                        row_ref[pl.ds(c * _LANES, _LANES)] = vals[t_i][c]

        def fire_store(j, s):
            pltpu.async_copy(
                rows_v.at[s],
                out_hbm.at[pl.ds(base + j * _SUPER, _SUPER)],
                ssem.at[s],
            )

        def wait_store(s):
            pltpu.make_async_copy(
                rows_v.at[s],
                out_hbm.at[pl.ds(base, _SUPER)],
                ssem.at[s],
            ).wait()

        def body(i, carry):
            for s in (0, 1):
                j = i * 2 + s

                @pl.when(j >= 2)
                def _():
                    wait_store(s)

                run_groups(j, s)
                fire_store(j, s)
            return carry

        lax.fori_loop(0, n_super // 2, body, 0)
        wait_store(0)
        wait_store(1)

    return k(table_flat, ts_flat)


def kernel(inputs, timestamp, W, b):
    batch, max_len = timestamp.shape
    emb = W.shape[0]
    table = (W.T + b[None, :]).reshape(-1)
    out = _sc_lookup(table, timestamp.reshape(batch * max_len), batch, max_len, emb)
    return out.reshape(batch, max_len, emb)
